# Initial kernel scaffold; baseline (speedup 1.0000x reference)
import jax
import jax.numpy as jnp
from jax.experimental import pallas as pl


def kernel(x, edge_index, W1, b1, W2, b2, Wp1, bp1, Wp2, bp2):
    raise NotImplementedError("write your pallas kernel here")


# speedup vs baseline: 5.6495x; 5.6495x over previous
import functools

import jax
import jax.numpy as jnp
from jax import lax
from jax.experimental import pallas as pl
from jax.experimental.pallas import tpu as pltpu
from jax.experimental.pallas import tpu_sc as plsc

_N = 10000
_E = 160000
_NPAD = 10112
_D = 128
_DH = 256
_TEMP = 0.4
_ER1, _ER2 = 0.2, 0.4
_FM1, _FM2 = 0.3, 0.4

_NC, _NS, _L = 2, 16, 16
_CH = 128
_NCH = 88
_EPAD = _NS * _NCH * _CH
_RPT = _NPAD // _NS
_NDUM = 112

_BS = 1000
_NB = _N // _BS
_RB = _NPAD // 8


@functools.lru_cache(maxsize=1)
def _sc_kernels():
    mesh = plsc.VectorSubcoreMesh(core_axis_name="c", subcore_axis_name="s",
                                  num_cores=_NC, num_subcores=_NS)

    @functools.partial(
        pl.kernel,
        out_type=jax.ShapeDtypeStruct((_NC * 2 * _NPAD, _D), jnp.float32),
        mesh=mesh,
        scratch_types=[
            pltpu.VMEM((_NCH, _CH), jnp.int32),
            pltpu.VMEM((_NCH, _CH), jnp.int32),
            pltpu.VMEM((_CH, _D), jnp.float32),
            pltpu.VMEM_SHARED((_NPAD, _D), jnp.float32),
        ],
    )
    def _deg_kernel(srcl_hbm, dstl_hbm, zeros_hbm, ones_hbm, out_hbm,
                    srcv, dstv, onesv, acc):
        c = lax.axis_index("c")
        s = lax.axis_index("s")
        w = c * _NS + s
        rows = pl.ds(s * _RPT, _RPT)
        pltpu.sync_copy(srcl_hbm.at[w], srcv)
        pltpu.sync_copy(dstl_hbm.at[w], dstv)
        pltpu.sync_copy(ones_hbm, onesv)
        for k, idxv in ((0, srcv), (1, dstv)):
            pltpu.sync_copy(zeros_hbm.at[rows], acc.at[rows])
            plsc.subcore_barrier()

            def body(j, carry, idxv=idxv):
                pltpu.sync_copy(onesv, acc.at[idxv.at[j]], add=True)
                return carry

            lax.fori_loop(0, _NCH, body, 0)
            plsc.subcore_barrier()
            pltpu.sync_copy(acc.at[rows],
                            out_hbm.at[pl.ds((c * 2 + k) * _NPAD + s * _RPT,
                                             _RPT)])
            plsc.subcore_barrier()

    @functools.partial(
        pl.kernel,
        out_type=jax.ShapeDtypeStruct((_NC * _NPAD, _D), jnp.float32),
        mesh=mesh,
        scratch_types=[
            pltpu.VMEM((_NCH, _CH), jnp.int32),
            pltpu.VMEM((_NCH, _CH), jnp.int32),
            pltpu.VMEM((_CH, _D), jnp.float32),
            pltpu.VMEM_SHARED((_NPAD, _D), jnp.float32),
            pltpu.SemaphoreType.DMA,
        ],
    )
    def _agg_kernel(table_hbm, srcg_hbm, dstl_hbm, zeros_hbm, out_hbm,
                    srcv, dstv, rows, acc, sem):
        c = lax.axis_index("c")
        s = lax.axis_index("s")
        w = c * _NS + s
        pltpu.sync_copy(srcg_hbm.at[w], srcv)
        pltpu.sync_copy(dstl_hbm.at[w], dstv)
        pltpu.sync_copy(zeros_hbm.at[pl.ds(s * _RPT, _RPT)],
                        acc.at[pl.ds(s * _RPT, _RPT)])
        plsc.subcore_barrier()

        def body(j, carry):
            pltpu.async_copy(table_hbm.at[srcv.at[j]], rows, sem).wait()
            pltpu.sync_copy(rows, acc.at[dstv.at[j]], add=True)
            return carry

        lax.fori_loop(0, _NCH, body, 0)
        plsc.subcore_barrier()
        pltpu.sync_copy(acc.at[pl.ds(s * _RPT, _RPT)],
                        out_hbm.at[pl.ds(c * _NPAD + s * _RPT, _RPT)])

    return _deg_kernel, _agg_kernel


def _prep_body(deg_ref, x_ref, colmul_ref, t1_ref, ns_ref, nd_ref):
    degs = deg_ref[0, 0][:, 0:1]
    degd = deg_ref[0, 1][:, 0:1]
    ns = jnp.where(degs > 0, lax.rsqrt(jnp.maximum(degs, 1e-12)), 0.0)
    nd = jnp.where(degd > 0, lax.rsqrt(jnp.maximum(degd, 1e-12)), 0.0)
    t1_ref[0] = x_ref[...] * colmul_ref[0] * ns
    ns_ref[0] = ns
    nd_ref[0] = nd


def _mid_body(agg_ref, W1_ref, b1_ref, W2_ref, ns_ref, nd_ref, t2_ref):
    r = pl.program_id(1)
    out1 = (jnp.dot(agg_ref[0], W1_ref[...],
                    preferred_element_type=jnp.float32)
            * nd_ref[0] + b1_ref[...])
    t2 = jnp.dot(out1 * ns_ref[0], W2_ref[...],
                 preferred_element_type=jnp.float32)
    valid = (r * _RB + lax.broadcasted_iota(jnp.int32, (_RB, 1), 0) < _N)
    t2_ref[0] = jnp.where(valid, t2, 0.0)


def _proj_body(agg_ref, nd_ref, b2_ref, Wp1_ref, bp1_ref, Wp2_ref, bp2_ref,
               pn_ref, sd_ref):
    z = agg_ref[0] * nd_ref[0] + b2_ref[...]
    h = jnp.dot(z, Wp1_ref[...], preferred_element_type=jnp.float32) + bp1_ref[...]
    h = jnp.where(h > 0, h, jnp.exp(jnp.minimum(h, 0.0)) - 1.0)
    p = jnp.dot(h, Wp2_ref[...], preferred_element_type=jnp.float32) + bp2_ref[...]
    nrm = jnp.maximum(jnp.sqrt(jnp.sum(p * p, axis=1, keepdims=True)), 1e-12)
    pn = p / nrm
    pn_ref[0] = pn
    sd_ref[0] = jnp.sum(pn * pn, axis=1, keepdims=True)


def _sim_body(p1i_ref, p1j_ref, p2i_ref, p2j_ref, s11_ref, s22_ref,
              out_ref, x1acc, x2acc, c21acc, d12acc):
    i = pl.program_id(0)
    j = pl.program_id(1)
    inv_t = 1.0 / _TEMP
    A1 = p1i_ref[0]
    B1 = p1j_ref[0]
    A2 = p2i_ref[0]
    B2 = p2j_ref[0]
    dims = (((1,), (1,)), ((), ()))
    S12 = lax.dot_general(A1, B2, dims, preferred_element_type=jnp.float32)
    E11 = jnp.exp(lax.dot_general(A1, B1, dims,
                                  preferred_element_type=jnp.float32) * inv_t)
    E22 = jnp.exp(lax.dot_general(A2, B2, dims,
                                  preferred_element_type=jnp.float32) * inv_t)
    E12 = jnp.exp(S12 * inv_t)

    ones = jnp.ones((1, _BS), jnp.float32)
    rdims = (((1,), (1,)), ((), ()))
    cdims = (((1,), (0,)), ((), ()))
    r1 = (lax.dot_general(ones, E11, rdims, preferred_element_type=jnp.float32)
          + lax.dot_general(ones, E12, rdims,
                            preferred_element_type=jnp.float32))
    r2 = lax.dot_general(ones, E22, rdims, preferred_element_type=jnp.float32)
    c21 = lax.dot_general(ones, E12, cdims,
                          preferred_element_type=jnp.float32)

    @pl.when(j == 0)
    def _():
        x1acc[i] = r1 - jnp.exp(s11_ref[0, 0] * inv_t)
        x2acc[i] = r2 - jnp.exp(s22_ref[0, 0] * inv_t)

    @pl.when(j != 0)
    def _():
        x1acc[i] += r1
        x2acc[i] += r2

    @pl.when(i == 0)
    def _():
        c21acc[j] = c21

    @pl.when(i != 0)
    def _():
        c21acc[j] += c21

    @pl.when(i == j)
    def _():
        eye = (lax.broadcasted_iota(jnp.int32, (_BS, _BS), 0)
               == lax.broadcasted_iota(jnp.int32, (_BS, _BS), 1))
        d12acc[i] = lax.dot_general(ones, jnp.where(eye, S12, 0.0), rdims,
                                    preferred_element_type=jnp.float32)

    @pl.when((i == _NB - 1) & (j == _NB - 1))
    def _():
        x2 = x2acc[...] + c21acc[...]
        l = (jnp.log(x1acc[...]) + jnp.log(x2)
             - 2.0 * inv_t * d12acc[...])
        out_ref[...] = (jnp.sum(l) * (0.5 / _N))[None, None]


def kernel(x, edge_index, W1, b1, W2, b2, Wp1, bp1, Wp2, bp2):
    f32 = jnp.float32
    kv = jax.random.key(42)
    k1, k2, k3, k4 = jax.random.split(kv, 4)
    keep1 = jax.random.bernoulli(k1, 1.0 - _ER1, (_E,))
    colmul1 = 1.0 - (jax.random.uniform(k2, (_D,)) < _FM1).astype(f32)
    keep2 = jax.random.bernoulli(k3, 1.0 - _ER2, (_E,))
    colmul2 = 1.0 - (jax.random.uniform(k4, (_D,)) < _FM2).astype(f32)

    loops = jnp.arange(_N, dtype=jnp.int32)
    dummy_e = _N + (jnp.arange(_E, dtype=jnp.int32) % _NDUM)
    npad_e = _EPAD - _E - _N
    pad = _N + (jnp.arange(npad_e, dtype=jnp.int32) % _NDUM)

    def prep(keep):
        src = jnp.where(keep, edge_index[0], dummy_e)
        dst = jnp.where(keep, edge_index[1], dummy_e)
        src = jnp.concatenate([src, loops, pad]).reshape(_NS, _NCH, _CH)
        dst = jnp.concatenate([dst, loops, pad]).reshape(_NS, _NCH, _CH)
        return src, dst

    s1, d1 = prep(keep1)
    s2, d2 = prep(keep2)
    srcl = jnp.stack([s1, s2]).reshape(_NC * _NS, _NCH, _CH)
    dstl = jnp.stack([d1, d2]).reshape(_NC * _NS, _NCH, _CH)
    off = (jnp.arange(_NC, dtype=jnp.int32) * _NPAD)[:, None, None, None]
    srcg = (jnp.stack([s1, s2]) + off).reshape(_NC * _NS, _NCH, _CH)

    onesd = jnp.ones((_CH, _D), f32)
    zerosd = jnp.zeros((_NPAD, _D), f32)

    _deg_kernel, _agg_kernel = _sc_kernels()
    degs = _deg_kernel(srcl, dstl, zerosd, onesd)
    degs = degs.reshape(_NC, 2, _NPAD, _D)[..., :16]

    xpad = jnp.zeros((_NPAD, _D), f32).at[:_N].set(x)
    colmul = jnp.stack([colmul1, colmul2]).reshape(_NC, 1, _D)

    table1, nsa, nda = pl.pallas_call(
        _prep_body,
        grid=(_NC, _NPAD // _RB),
        in_specs=[
            pl.BlockSpec((1, 2, _RB, 16), lambda v, r: (v, 0, r, 0)),
            pl.BlockSpec((_RB, _D), lambda v, r: (r, 0)),
            pl.BlockSpec((1, 1, _D), lambda v, r: (v, 0, 0)),
        ],
        out_specs=[
            pl.BlockSpec((1, _RB, _D), lambda v, r: (v, r, 0)),
            pl.BlockSpec((1, _RB, 1), lambda v, r: (v, r, 0)),
            pl.BlockSpec((1, _RB, 1), lambda v, r: (v, r, 0)),
        ],
        out_shape=[
            jax.ShapeDtypeStruct((_NC, _NPAD, _D), f32),
            jax.ShapeDtypeStruct((_NC, _NPAD, 1), f32),
            jax.ShapeDtypeStruct((_NC, _NPAD, 1), f32),
        ],
    )(degs, xpad, colmul)

    agg1 = _agg_kernel(table1.reshape(_NC * _NPAD, _D), srcg, dstl, zerosd)

    table2 = pl.pallas_call(
        _mid_body,
        grid=(_NC, _NPAD // _RB),
        in_specs=[
            pl.BlockSpec((1, _RB, _D), lambda v, r: (v, r, 0)),
            pl.BlockSpec((_D, _DH), lambda v, r: (0, 0)),
            pl.BlockSpec((1, _DH), lambda v, r: (0, 0)),
            pl.BlockSpec((_DH, _D), lambda v, r: (0, 0)),
            pl.BlockSpec((1, _RB, 1), lambda v, r: (v, r, 0)),
            pl.BlockSpec((1, _RB, 1), lambda v, r: (v, r, 0)),
        ],
        out_specs=pl.BlockSpec((1, _RB, _D), lambda v, r: (v, r, 0)),
        out_shape=jax.ShapeDtypeStruct((_NC, _NPAD, _D), f32),
    )(agg1.reshape(_NC, _NPAD, _D), W1, b1.reshape(1, _DH), W2, nsa, nda)

    agg2 = _agg_kernel(table2.reshape(_NC * _NPAD, _D), srcg, dstl, zerosd)

    pn, sd = pl.pallas_call(
        _proj_body,
        grid=(_NC, _N // _BS),
        in_specs=[
            pl.BlockSpec((1, _BS, _D), lambda v, r: (v, r, 0)),
            pl.BlockSpec((1, _BS, 1), lambda v, r: (v, r, 0)),
            pl.BlockSpec((1, _D), lambda v, r: (0, 0)),
            pl.BlockSpec((_D, _D), lambda v, r: (0, 0)),
            pl.BlockSpec((1, _D), lambda v, r: (0, 0)),
            pl.BlockSpec((_D, _D), lambda v, r: (0, 0)),
            pl.BlockSpec((1, _D), lambda v, r: (0, 0)),
        ],
        out_specs=[
            pl.BlockSpec((1, _BS, _D), lambda v, r: (v, r, 0)),
            pl.BlockSpec((1, _BS, 1), lambda v, r: (v, r, 0)),
        ],
        out_shape=[
            jax.ShapeDtypeStruct((_NC, _N, _D), f32),
            jax.ShapeDtypeStruct((_NC, _N, 1), f32),
        ],
    )(agg2.reshape(_NC, _NPAD, _D), nda, b2.reshape(1, _D),
      Wp1, bp1.reshape(1, _D), Wp2, bp2.reshape(1, _D))

    loss = pl.pallas_call(
        _sim_body,
        grid=(_NB, _NB),
        in_specs=[
            pl.BlockSpec((1, _BS, _D), lambda i, j: (0, i, 0)),
            pl.BlockSpec((1, _BS, _D), lambda i, j: (0, j, 0)),
            pl.BlockSpec((1, _BS, _D), lambda i, j: (1, i, 0)),
            pl.BlockSpec((1, _BS, _D), lambda i, j: (1, j, 0)),
            pl.BlockSpec((1, 1, 1, _BS), lambda i, j: (0, i, 0, 0)),
            pl.BlockSpec((1, 1, 1, _BS), lambda i, j: (1, i, 0, 0)),
        ],
        out_specs=pl.BlockSpec((1, 1), lambda i, j: (0, 0)),
        out_shape=jax.ShapeDtypeStruct((1, 1), f32),
        scratch_shapes=[
            pltpu.VMEM((_NB, 1, _BS), f32),
            pltpu.VMEM((_NB, 1, _BS), f32),
            pltpu.VMEM((_NB, 1, _BS), f32),
            pltpu.VMEM((_NB, 1, _BS), f32),
        ],
    )(pn, pn, pn, pn, sd.reshape(_NC, _NB, 1, _BS), sd.reshape(_NC, _NB, 1, _BS))

    return loss[0, 0]


# speedup vs baseline: 6.3644x; 1.1265x over previous
import functools

import jax
import jax.numpy as jnp
from jax import lax
from jax.experimental import pallas as pl
from jax.experimental.pallas import tpu as pltpu
from jax.experimental.pallas import tpu_sc as plsc

_N = 10000
_E = 160000
_NPAD = 10112
_D = 128
_DH = 256
_TEMP = 0.4
_ER1, _ER2 = 0.2, 0.4
_FM1, _FM2 = 0.3, 0.4

_NC, _NS, _L = 2, 16, 16
_CH = 128
_NCH = 88
_EPAD = _NS * _NCH * _CH
_NCHA = 96
_HCH = 48
_EPADA = _NS * _NCHA * _CH
_RPT = _NPAD // _NS
_NDUM = 112

_BS = 1000
_NB = _N // _BS
_RB = _NPAD // 8


@functools.lru_cache(maxsize=1)
def _sc_kernels():
    mesh = plsc.VectorSubcoreMesh(core_axis_name="c", subcore_axis_name="s",
                                  num_cores=_NC, num_subcores=_NS)

    @functools.partial(
        pl.kernel,
        out_type=jax.ShapeDtypeStruct((_NC * 2 * _NPAD, _D), jnp.float32),
        mesh=mesh,
        scratch_types=[
            pltpu.VMEM((_NCH, _CH), jnp.int32),
            pltpu.VMEM((_NCH, _CH), jnp.int32),
            pltpu.VMEM((_CH, _D), jnp.float32),
            pltpu.VMEM_SHARED((_NPAD, _D), jnp.float32),
        ],
    )
    def _deg_kernel(srcl_hbm, dstl_hbm, zeros_hbm, ones_hbm, out_hbm,
                    srcv, dstv, onesv, acc):
        c = lax.axis_index("c")
        s = lax.axis_index("s")
        w = c * _NS + s
        rows = pl.ds(s * _RPT, _RPT)
        pltpu.sync_copy(srcl_hbm.at[w], srcv)
        pltpu.sync_copy(dstl_hbm.at[w], dstv)
        pltpu.sync_copy(ones_hbm, onesv)
        for k, idxv in ((0, srcv), (1, dstv)):
            pltpu.sync_copy(zeros_hbm.at[rows], acc.at[rows])
            plsc.subcore_barrier()

            def body(j, carry, idxv=idxv):
                pltpu.sync_copy(onesv, acc.at[idxv.at[j]], add=True)
                return carry

            lax.fori_loop(0, _NCH, body, 0)
            plsc.subcore_barrier()
            pltpu.sync_copy(acc.at[rows],
                            out_hbm.at[pl.ds((c * 2 + k) * _NPAD + s * _RPT,
                                             _RPT)])
            plsc.subcore_barrier()

    @functools.partial(
        pl.kernel,
        out_type=jax.ShapeDtypeStruct((_NC * _NPAD, _D), jnp.float32),
        mesh=mesh,
        scratch_types=[
            pltpu.VMEM((_HCH, _CH), jnp.int32),
            pltpu.VMEM((_HCH, _CH), jnp.int32),
            pltpu.VMEM((_CH, _D), jnp.float32),
            pltpu.VMEM((_CH, _D), jnp.float32),
            pltpu.VMEM_SHARED((_NPAD, _D), jnp.float32),
            pltpu.SemaphoreType.DMA,
            pltpu.SemaphoreType.DMA,
        ],
    )
    def _agg_kernel(table_hbm, srcg_hbm, dstl_hbm, zeros_hbm, out_hbm,
                    srcv, dstv, rows_a, rows_b, acc, sem_a, sem_b):
        c = lax.axis_index("c")
        s = lax.axis_index("s")
        w = c * _NS + s
        pltpu.sync_copy(zeros_hbm.at[pl.ds(s * _RPT, _RPT)],
                        acc.at[pl.ds(s * _RPT, _RPT)])
        plsc.subcore_barrier()

        for h in range(2):
            pltpu.sync_copy(srcg_hbm.at[2 * w + h], srcv)
            pltpu.sync_copy(dstl_hbm.at[2 * w + h], dstv)
            pltpu.async_copy(table_hbm.at[srcv.at[0]], rows_a, sem_a)

            def body(t, carry):
                j = 2 * t
                pltpu.async_copy(table_hbm.at[srcv.at[j + 1]], rows_b, sem_b)
                pltpu.make_async_copy(table_hbm.at[srcv.at[j]], rows_a,
                                      sem_a).wait()
                pltpu.sync_copy(rows_a, acc.at[dstv.at[j]], add=True)

                @pl.when(j + 2 < _HCH)
                def _():
                    pltpu.async_copy(table_hbm.at[srcv.at[j + 2]], rows_a,
                                     sem_a)

                pltpu.make_async_copy(table_hbm.at[srcv.at[j + 1]], rows_b,
                                      sem_b).wait()
                pltpu.sync_copy(rows_b, acc.at[dstv.at[j + 1]], add=True)
                return carry

            lax.fori_loop(0, _HCH // 2, body, 0)
        plsc.subcore_barrier()
        pltpu.sync_copy(acc.at[pl.ds(s * _RPT, _RPT)],
                        out_hbm.at[pl.ds(c * _NPAD + s * _RPT, _RPT)])

    return _deg_kernel, _agg_kernel


def _prep_body(deg_ref, x_ref, colmul_ref, t1_ref, ns_ref, nd_ref):
    degs = deg_ref[0, 0][:, 0:1]
    degd = deg_ref[0, 1][:, 0:1]
    ns = jnp.where(degs > 0, lax.rsqrt(jnp.maximum(degs, 1e-12)), 0.0)
    nd = jnp.where(degd > 0, lax.rsqrt(jnp.maximum(degd, 1e-12)), 0.0)
    t1_ref[0] = x_ref[...] * colmul_ref[0] * ns
    ns_ref[0] = ns
    nd_ref[0] = nd


def _mid_body(agg_ref, W1_ref, b1_ref, W2_ref, ns_ref, nd_ref, t2_ref):
    r = pl.program_id(1)
    out1 = (jnp.dot(agg_ref[0], W1_ref[...],
                    preferred_element_type=jnp.float32)
            * nd_ref[0] + b1_ref[...])
    t2 = jnp.dot(out1 * ns_ref[0], W2_ref[...],
                 preferred_element_type=jnp.float32)
    valid = (r * _RB + lax.broadcasted_iota(jnp.int32, (_RB, 1), 0) < _N)
    t2_ref[0] = jnp.where(valid, t2, 0.0)


def _proj_body(agg_ref, nd_ref, b2_ref, Wp1_ref, bp1_ref, Wp2_ref, bp2_ref,
               pn_ref, sd_ref):
    z = agg_ref[0] * nd_ref[0] + b2_ref[...]
    h = jnp.dot(z, Wp1_ref[...], preferred_element_type=jnp.float32) + bp1_ref[...]
    h = jnp.where(h > 0, h, jnp.exp(jnp.minimum(h, 0.0)) - 1.0)
    p = jnp.dot(h, Wp2_ref[...], preferred_element_type=jnp.float32) + bp2_ref[...]
    nrm = jnp.maximum(jnp.sqrt(jnp.sum(p * p, axis=1, keepdims=True)), 1e-12)
    pn = p / nrm
    pn_ref[0] = pn
    sd_ref[0] = jnp.sum(pn * pn, axis=1, keepdims=True)


def _sim_body(p1i_ref, p1j_ref, p2i_ref, p2j_ref, s11_ref, s22_ref,
              out_ref, x1acc, x2acc, c21acc, d12acc):
    i = pl.program_id(0)
    j = pl.program_id(1)
    inv_t = 1.0 / _TEMP
    A1 = p1i_ref[0]
    B1 = p1j_ref[0]
    A2 = p2i_ref[0]
    B2 = p2j_ref[0]
    dims = (((1,), (1,)), ((), ()))
    ones = jnp.ones((1, _BS), jnp.float32)
    rdims = (((1,), (1,)), ((), ()))
    cdims = (((1,), (0,)), ((), ()))

    def rsum(M):
        return lax.dot_general(ones, M, rdims, preferred_element_type=jnp.float32)

    def csum(M):
        return lax.dot_general(ones, M, cdims, preferred_element_type=jnp.float32)

    S12 = lax.dot_general(A1, B2, dims, preferred_element_type=jnp.float32)
    E12 = jnp.exp(S12 * inv_t)

    @pl.when((i == 0) & (j == 0))
    def _():
        z = jnp.zeros((_NB, 1, _BS), jnp.float32)
        x1acc[...] = z
        x2acc[...] = z
        c21acc[...] = z
        d12acc[...] = z

    x1acc[i] += rsum(E12)
    c21acc[j] += csum(E12)

    @pl.when(j >= i)
    def _():
        E11 = jnp.exp(lax.dot_general(A1, B1, dims,
                                      preferred_element_type=jnp.float32) * inv_t)
        E22 = jnp.exp(lax.dot_general(A2, B2, dims,
                                      preferred_element_type=jnp.float32) * inv_t)
        x1acc[i] += rsum(E11)
        x2acc[i] += rsum(E22)

        @pl.when(j > i)
        def _():
            x1acc[j] += csum(E11)
            x2acc[j] += csum(E22)

    @pl.when(i == j)
    def _():
        eye = (lax.broadcasted_iota(jnp.int32, (_BS, _BS), 0)
               == lax.broadcasted_iota(jnp.int32, (_BS, _BS), 1))
        d12acc[i] = rsum(jnp.where(eye, S12, 0.0))
        x1acc[i] += -jnp.exp(s11_ref[0, 0] * inv_t)
        x2acc[i] += -jnp.exp(s22_ref[0, 0] * inv_t)

    @pl.when((i == _NB - 1) & (j == _NB - 1))
    def _():
        x2 = x2acc[...] + c21acc[...]
        l = (jnp.log(x1acc[...]) + jnp.log(x2)
             - 2.0 * inv_t * d12acc[...])
        out_ref[...] = (jnp.sum(l) * (0.5 / _N))[None, None]


def kernel(x, edge_index, W1, b1, W2, b2, Wp1, bp1, Wp2, bp2):
    f32 = jnp.float32
    kv = jax.random.key(42)
    k1, k2, k3, k4 = jax.random.split(kv, 4)
    keep1 = jax.random.bernoulli(k1, 1.0 - _ER1, (_E,))
    colmul1 = 1.0 - (jax.random.uniform(k2, (_D,)) < _FM1).astype(f32)
    keep2 = jax.random.bernoulli(k3, 1.0 - _ER2, (_E,))
    colmul2 = 1.0 - (jax.random.uniform(k4, (_D,)) < _FM2).astype(f32)

    loops = jnp.arange(_N, dtype=jnp.int32)
    dummy_e = _N + (jnp.arange(_E, dtype=jnp.int32) % _NDUM)
    npad_e = _EPAD - _E - _N
    pad = _N + (jnp.arange(npad_e, dtype=jnp.int32) % _NDUM)
    npad_a = _EPADA - _E - _N
    pada = _N + (jnp.arange(npad_a, dtype=jnp.int32) % _NDUM)

    def prep(keep):
        src = jnp.where(keep, edge_index[0], dummy_e)
        dst = jnp.where(keep, edge_index[1], dummy_e)
        srcd = jnp.concatenate([src, loops, pad]).reshape(_NS, _NCH, _CH)
        dstd = jnp.concatenate([dst, loops, pad]).reshape(_NS, _NCH, _CH)
        srca = jnp.concatenate([src, loops, pada]).reshape(_NS, 2, _HCH, _CH)
        dsta = jnp.concatenate([dst, loops, pada]).reshape(_NS, 2, _HCH, _CH)
        return srcd, dstd, srca, dsta

    s1, d1, sa1, da1 = prep(keep1)
    s2, d2, sa2, da2 = prep(keep2)
    srcl = jnp.stack([s1, s2]).reshape(_NC * _NS, _NCH, _CH)
    dstl = jnp.stack([d1, d2]).reshape(_NC * _NS, _NCH, _CH)
    off = (jnp.arange(_NC, dtype=jnp.int32) * _NPAD)[:, None, None, None, None]
    srcg = (jnp.stack([sa1, sa2]) + off).reshape(_NC * _NS * 2, _HCH, _CH)
    dsta = jnp.stack([da1, da2]).reshape(_NC * _NS * 2, _HCH, _CH)

    onesd = jnp.ones((_CH, _D), f32)
    zerosd = jnp.zeros((_NPAD, _D), f32)

    _deg_kernel, _agg_kernel = _sc_kernels()
    degs = _deg_kernel(srcl, dstl, zerosd, onesd).reshape(_NC, 2, _NPAD, _D)

    xpad = jnp.zeros((_NPAD, _D), f32).at[:_N].set(x)
    colmul = jnp.stack([colmul1, colmul2]).reshape(_NC, 1, _D)

    table1, nsa, nda = pl.pallas_call(
        _prep_body,
        grid=(_NC, _NPAD // _RB),
        in_specs=[
            pl.BlockSpec((1, 2, _RB, _D), lambda v, r: (v, 0, r, 0)),
            pl.BlockSpec((_RB, _D), lambda v, r: (r, 0)),
            pl.BlockSpec((1, 1, _D), lambda v, r: (v, 0, 0)),
        ],
        out_specs=[
            pl.BlockSpec((1, _RB, _D), lambda v, r: (v, r, 0)),
            pl.BlockSpec((1, _RB, 1), lambda v, r: (v, r, 0)),
            pl.BlockSpec((1, _RB, 1), lambda v, r: (v, r, 0)),
        ],
        out_shape=[
            jax.ShapeDtypeStruct((_NC, _NPAD, _D), f32),
            jax.ShapeDtypeStruct((_NC, _NPAD, 1), f32),
            jax.ShapeDtypeStruct((_NC, _NPAD, 1), f32),
        ],
    )(degs, xpad, colmul)

    agg1 = _agg_kernel(table1.reshape(_NC * _NPAD, _D), srcg, dsta, zerosd)

    table2 = pl.pallas_call(
        _mid_body,
        grid=(_NC, _NPAD // _RB),
        in_specs=[
            pl.BlockSpec((1, _RB, _D), lambda v, r: (v, r, 0)),
            pl.BlockSpec((_D, _DH), lambda v, r: (0, 0)),
            pl.BlockSpec((1, _DH), lambda v, r: (0, 0)),
            pl.BlockSpec((_DH, _D), lambda v, r: (0, 0)),
            pl.BlockSpec((1, _RB, 1), lambda v, r: (v, r, 0)),
            pl.BlockSpec((1, _RB, 1), lambda v, r: (v, r, 0)),
        ],
        out_specs=pl.BlockSpec((1, _RB, _D), lambda v, r: (v, r, 0)),
        out_shape=jax.ShapeDtypeStruct((_NC, _NPAD, _D), f32),
    )(agg1.reshape(_NC, _NPAD, _D), W1, b1.reshape(1, _DH), W2, nsa, nda)

    agg2 = _agg_kernel(table2.reshape(_NC * _NPAD, _D), srcg, dsta, zerosd)

    pn, sd = pl.pallas_call(
        _proj_body,
        grid=(_NC, _N // _BS),
        in_specs=[
            pl.BlockSpec((1, _BS, _D), lambda v, r: (v, r, 0)),
            pl.BlockSpec((1, _BS, 1), lambda v, r: (v, r, 0)),
            pl.BlockSpec((1, _D), lambda v, r: (0, 0)),
            pl.BlockSpec((_D, _D), lambda v, r: (0, 0)),
            pl.BlockSpec((1, _D), lambda v, r: (0, 0)),
            pl.BlockSpec((_D, _D), lambda v, r: (0, 0)),
            pl.BlockSpec((1, _D), lambda v, r: (0, 0)),
        ],
        out_specs=[
            pl.BlockSpec((1, _BS, _D), lambda v, r: (v, r, 0)),
            pl.BlockSpec((1, _BS, 1), lambda v, r: (v, r, 0)),
        ],
        out_shape=[
            jax.ShapeDtypeStruct((_NC, _N, _D), f32),
            jax.ShapeDtypeStruct((_NC, _N, 1), f32),
        ],
    )(agg2.reshape(_NC, _NPAD, _D), nda, b2.reshape(1, _D),
      Wp1, bp1.reshape(1, _D), Wp2, bp2.reshape(1, _D))

    loss = pl.pallas_call(
        _sim_body,
        grid=(_NB, _NB),
        in_specs=[
            pl.BlockSpec((1, _BS, _D), lambda i, j: (0, i, 0)),
            pl.BlockSpec((1, _BS, _D), lambda i, j: (0, j, 0)),
            pl.BlockSpec((1, _BS, _D), lambda i, j: (1, i, 0)),
            pl.BlockSpec((1, _BS, _D), lambda i, j: (1, j, 0)),
            pl.BlockSpec((1, 1, 1, _BS), lambda i, j: (0, i, 0, 0)),
            pl.BlockSpec((1, 1, 1, _BS), lambda i, j: (1, i, 0, 0)),
        ],
        out_specs=pl.BlockSpec((1, 1), lambda i, j: (0, 0)),
        out_shape=jax.ShapeDtypeStruct((1, 1), f32),
        scratch_shapes=[
            pltpu.VMEM((_NB, 1, _BS), f32),
            pltpu.VMEM((_NB, 1, _BS), f32),
            pltpu.VMEM((_NB, 1, _BS), f32),
            pltpu.VMEM((_NB, 1, _BS), f32),
        ],
    )(pn, pn, pn, pn, sd.reshape(_NC, _NB, 1, _BS), sd.reshape(_NC, _NB, 1, _BS))

    return loss[0, 0]
